# Initial kernel scaffold; baseline (speedup 1.0000x reference)
import jax
import jax.numpy as jnp
from jax.experimental import pallas as pl


def kernel(x, token_table, pos_table):
    raise NotImplementedError("write your pallas kernel here")


# speedup vs baseline: 3.2631x; 3.2631x over previous
import functools

import jax
import jax.numpy as jnp
from jax import lax
from jax.experimental import pallas as pl
from jax.experimental.pallas import tpu as pltpu
from jax.experimental.pallas import tpu_sc as plsc

_VOCAB = 10000
_MAXLEN = 150
_DIM = 32
_BATCH = 4096

_N = _BATCH * _MAXLEN
_NW = 32
_SUB = 120
_NSUB = 5
_CHUNK = _SUB * _NSUB
_NCHUNK = _N // _CHUNK
_CPW = _NCHUNK // _NW
_REPS = _CHUNK // _MAXLEN
_HALF = _DIM // 16


def _emb_body(x_hbm, tok_hbm, pos_hbm, out_hbm, idx_v, rows_v, pos_v, sem):
    wid = lax.axis_index("s") * 2 + lax.axis_index("c")

    pltpu.sync_copy(pos_hbm, pos_v)

    def chunk_body(c, _):
        g = wid * _CPW + c
        pltpu.sync_copy(x_hbm.at[g], idx_v)
        copies = [
            pltpu.async_copy(
                tok_hbm.at[idx_v.at[j]],
                rows_v.at[pl.ds(j * _SUB, _SUB)],
                sem,
            )
            for j in range(_NSUB)
        ]
        for cp in copies:
            cp.wait()

        for rep in range(_REPS):
            base = rep * _MAXLEN

            def add_body(p, _, base=base):
                r = base + p
                for h in range(_HALF):
                    sl = pl.ds(h * 16, 16)
                    rows_v[r, sl] = rows_v[r, sl] + pos_v[p, sl]
                return 0

            lax.fori_loop(0, _MAXLEN, add_body, 0)

        pltpu.sync_copy(rows_v, out_hbm.at[pl.ds(g * _CHUNK, _CHUNK)])
        return 0

    lax.fori_loop(0, _CPW, chunk_body, 0)


@jax.jit
def _emb_call(x3d, token_table, pos_table):
    mesh = plsc.VectorSubcoreMesh(core_axis_name="c", subcore_axis_name="s")
    k = functools.partial(
        pl.kernel,
        mesh=mesh,
        out_type=jax.ShapeDtypeStruct((_N, _DIM), jnp.float32),
        scratch_types=[
            pltpu.VMEM((_NSUB, _SUB), jnp.int32),
            pltpu.VMEM((_CHUNK, _DIM), jnp.float32),
            pltpu.VMEM((_MAXLEN, _DIM), jnp.float32),
            pltpu.SemaphoreType.DMA,
        ],
        compiler_params=pltpu.CompilerParams(use_tc_tiling_on_sc=False),
    )(_emb_body)
    return k(x3d, token_table, pos_table)


def kernel(x, token_table, pos_table):
    x3d = x.astype(jnp.int32).reshape(_NCHUNK, _NSUB, _SUB)
    out = _emb_call(x3d, token_table, pos_table)
    return out.reshape(_BATCH, _MAXLEN, _DIM)


# speedup vs baseline: 5.7640x; 1.7665x over previous
import functools

import jax
import jax.numpy as jnp
from jax import lax
from jax.experimental import pallas as pl
from jax.experimental.pallas import tpu as pltpu
from jax.experimental.pallas import tpu_sc as plsc

_VOCAB = 10000
_MAXLEN = 150
_DIM = 32
_BATCH = 4096

_N = _BATCH * _MAXLEN
_NW = 32
_SUB = 120
_NSUB = 5
_CHUNK = _SUB * _NSUB
_NCHUNK = _N // _CHUNK
_CPW = _NCHUNK // _NW
_REPS = _CHUNK // _MAXLEN
_HALF = _DIM // 16


def _emb_body(x_hbm, tok_hbm, pos_hbm, out_hbm, idx_v, rows_v, wrows_v, pos_v, sem):
    wid = lax.axis_index("s") * 2 + lax.axis_index("c")

    pltpu.sync_copy(pos_hbm, pos_v)

    def chunk_body(c, _):
        g = wid * _CPW + c
        pltpu.sync_copy(x_hbm.at[pl.ds(g * _CHUNK, _CHUNK)], idx_v)
        copies = [
            pltpu.async_copy(
                tok_hbm.at[idx_v.at[pl.ds(j * _SUB, _SUB)]],
                rows_v.at[pl.ds(j * _SUB, _SUB)],
                sem,
            )
            for j in range(_NSUB)
        ]
        for cp in copies:
            cp.wait()

        for rep in range(_REPS):
            base = rep * _MAXLEN

            def add_body(p, _, base=base):
                r = base + p
                for h in range(_HALF):
                    sl = pl.ds(h * 16, 16)
                    wrows_v[pl.ds(r * _DIM + h * 16, 16)] = (
                        rows_v[r, sl] + pos_v[p, sl]
                    )
                return 0

            lax.fori_loop(0, _MAXLEN, add_body, 0)

        pltpu.sync_copy(
            wrows_v, out_hbm.at[pl.ds(g * _CHUNK * _DIM, _CHUNK * _DIM)]
        )
        return 0

    lax.fori_loop(0, _CPW, chunk_body, 0)


@jax.jit
def _emb_call(x_flat, token_table, pos_table):
    mesh = plsc.VectorSubcoreMesh(core_axis_name="c", subcore_axis_name="s")
    k = functools.partial(
        pl.kernel,
        mesh=mesh,
        out_type=jax.ShapeDtypeStruct((_N * _DIM,), jnp.float32),
        scratch_types=[
            pltpu.VMEM((_CHUNK,), jnp.int32),
            pltpu.VMEM((_CHUNK, _DIM), jnp.float32),
            pltpu.VMEM((_CHUNK * _DIM,), jnp.float32),
            pltpu.VMEM((_MAXLEN, _DIM), jnp.float32),
            pltpu.SemaphoreType.DMA,
        ],
        compiler_params=pltpu.CompilerParams(use_tc_tiling_on_sc=False),
    )(_emb_body)
    return k(x_flat, token_table, pos_table)


def kernel(x, token_table, pos_table):
    x_flat = x.astype(jnp.int32).reshape(_N)
    out = _emb_call(x_flat, token_table, pos_table)
    return out.reshape(_BATCH, _MAXLEN, _DIM)
